# baseline (speedup 1.0000x reference)
import functools

import jax
import jax.numpy as jnp
from jax import lax
from jax.scipy.linalg import block_diag
from jax.experimental import pallas as pl
from jax.experimental.pallas import tpu as pltpu
from jax.experimental.pallas import tpu_sc as plsc

F32 = jnp.float32
BF16 = jnp.bfloat16

IN_NF_ = 16
OUT_NF_ = 12
EF_KEY_ = 120
OUT_CEF_ = 4

TE = 1000
TN = 1000


def _leaky(x):
    return jnp.where(x > 0, x, 0.2 * x)


def _pad_rows(w, rows):
    return jnp.pad(w, ((0, rows - w.shape[0]), (0, 0)))


_NC, _NS, _NW = 2, 16, 32


def _sc_mesh():
    return plsc.VectorSubcoreMesh(core_axis_name="c", subcore_axis_name="s")


_SC_PARAMS = pltpu.CompilerParams(use_tc_tiling_on_sc=False,
                                  needs_layout_passes=False)


def _sc_scatter_add(rows, idx, zeros, n_nodes, block):
    E = rows.shape[0]
    NB = E // block
    nb_per_w = (NB + _NW - 1) // _NW
    rows_seg = n_nodes // _NS

    @functools.partial(
        pl.kernel, mesh=_sc_mesh(), compiler_params=_SC_PARAMS,
        out_type=jax.ShapeDtypeStruct((_NC, _NS, rows_seg, 16), F32),
        scratch_types=[
            pltpu.VMEM((block,), jnp.int32),
            pltpu.VMEM((block, 16), F32),
            pltpu.VMEM_SHARED((n_nodes, 16), F32),
        ],
    )
    def k(rows_hbm, idx_hbm, zeros_hbm, out_hbm, idx_v, rows_v, acc_s):
        c = lax.axis_index("c")
        s = lax.axis_index("s")
        wid = s * _NC + c
        pltpu.sync_copy(zeros_hbm.at[s],
                        acc_s.at[pl.ds(s * rows_seg, rows_seg)])
        plsc.subcore_barrier()

        def body(i, _):
            j = i * _NW + wid

            @pl.when(j < NB)
            def _():
                base = j * block
                pltpu.sync_copy(idx_hbm.at[pl.ds(base, block)], idx_v)
                pltpu.sync_copy(rows_hbm.at[pl.ds(base, block)], rows_v)
                pltpu.sync_copy(rows_v, acc_s.at[idx_v], add=True)

            return 0

        lax.fori_loop(0, nb_per_w, body, 0)
        plsc.subcore_barrier()
        pltpu.sync_copy(acc_s.at[pl.ds(s * rows_seg, rows_seg)],
                        out_hbm.at[c, s])

    out = k(rows, idx, zeros.reshape(_NS, rows_seg, 16))
    return out.reshape(_NC, n_nodes, 16)


def _sc_scatter_add_pipe(rows, idx, zeros, n_nodes, block):
    E = rows.shape[0]
    nb = E // (_NW * block)
    rows_seg = n_nodes // _NS

    @functools.partial(
        pl.kernel, mesh=_sc_mesh(), compiler_params=_SC_PARAMS,
        out_type=jax.ShapeDtypeStruct((_NC, _NS, rows_seg, 16), F32),
        scratch_types=[
            pltpu.VMEM((2, block), jnp.int32),
            pltpu.VMEM((2, block, 16), F32),
            pltpu.VMEM_SHARED((n_nodes, 16), F32),
        ] + [pltpu.SemaphoreType.DMA] * 6,
    )
    def k(rows_hbm, idx_hbm, zeros_hbm, out_hbm, idx_v, rows_v, acc_s, *sems):
        si = sems[0:2]
        sr = sems[2:4]
        ss = sems[4:6]
        c = lax.axis_index("c")
        s = lax.axis_index("s")
        wid = s * _NC + c
        pltpu.sync_copy(zeros_hbm.at[s],
                        acc_s.at[pl.ds(s * rows_seg, rows_seg)])
        plsc.subcore_barrier()

        def base(i):
            return wid * (nb * block) + i * block

        loads = [None, None]
        scat = [None, None]
        loads[0] = (
            pltpu.async_copy(idx_hbm.at[pl.ds(base(0), block)],
                             idx_v.at[0], si[0]),
            pltpu.async_copy(rows_hbm.at[pl.ds(base(0), block)],
                             rows_v.at[0], sr[0]),
        )
        for i in range(nb):
            sl = i % 2
            nsl = (i + 1) % 2
            loads[sl][0].wait()
            loads[sl][1].wait()
            scat[sl] = pltpu.async_copy(rows_v.at[sl],
                                        acc_s.at[idx_v.at[sl]], ss[sl],
                                        add=True)
            if i + 1 < nb:
                if scat[nsl] is not None:
                    scat[nsl].wait()
                loads[nsl] = (
                    pltpu.async_copy(idx_hbm.at[pl.ds(base(i + 1), block)],
                                     idx_v.at[nsl], si[nsl]),
                    pltpu.async_copy(rows_hbm.at[pl.ds(base(i + 1), block)],
                                     rows_v.at[nsl], sr[nsl]),
                )
        scat[(nb - 1) % 2].wait()
        plsc.subcore_barrier()
        pltpu.sync_copy(acc_s.at[pl.ds(s * rows_seg, rows_seg)],
                        out_hbm.at[c, s])

    out = k(rows, idx, zeros.reshape(_NS, rows_seg, 16))
    return out.reshape(_NC, n_nodes, 16)


def _sc_edge_gather(tbl, src, dst, block):
    E = src.shape[0]
    NB = E // block
    nb_per_w = (NB + _NW - 1) // _NW

    @functools.partial(
        pl.kernel, mesh=_sc_mesh(), compiler_params=_SC_PARAMS,
        out_type=[
            jax.ShapeDtypeStruct((E, 32), BF16),
            jax.ShapeDtypeStruct((E, 32), BF16),
        ],
        scratch_types=[
            pltpu.VMEM((block,), jnp.int32),
            pltpu.VMEM((block, 32), BF16),
            pltpu.VMEM((block, 32), BF16),
        ],
    )
    def k(tbl_hbm, src_hbm, dst_hbm, xs_hbm, xd_hbm, idx_v, ra_v, rb_v):
        c = lax.axis_index("c")
        s = lax.axis_index("s")
        wid = s * _NC + c

        def body(i, _):
            j = i * _NW + wid

            @pl.when(j < NB)
            def _():
                base = j * block
                pltpu.sync_copy(src_hbm.at[pl.ds(base, block)], idx_v)
                pltpu.sync_copy(tbl_hbm.at[idx_v], ra_v)
                pltpu.sync_copy(ra_v, xs_hbm.at[pl.ds(base, block)])
                pltpu.sync_copy(dst_hbm.at[pl.ds(base, block)], idx_v)
                pltpu.sync_copy(tbl_hbm.at[idx_v], rb_v)
                pltpu.sync_copy(rb_v, xd_hbm.at[pl.ds(base, block)])

            return 0

        lax.fori_loop(0, nb_per_w, body, 0)

    return k(tbl, src, dst)


def _sc_edge_gather_pipe(tbl, src, dst, block):
    E = src.shape[0]
    nb = E // (_NW * block)

    @functools.partial(
        pl.kernel, mesh=_sc_mesh(), compiler_params=_SC_PARAMS,
        out_type=[
            jax.ShapeDtypeStruct((E, 32), BF16),
            jax.ShapeDtypeStruct((E, 32), BF16),
        ],
        scratch_types=[
            pltpu.VMEM((2, block), jnp.int32),
            pltpu.VMEM((2, block), jnp.int32),
            pltpu.VMEM((2, block, 32), BF16),
            pltpu.VMEM((2, block, 32), BF16),
        ] + [pltpu.SemaphoreType.DMA] * 12,
    )
    def k(tbl_hbm, src_hbm, dst_hbm, xs_hbm, xd_hbm,
          isv, idv, ra_v, rb_v, *sems):
        ia = sems[0:2]
        ib = sems[2:4]
        ga = sems[4:6]
        gb = sems[6:8]
        wa = sems[8:10]
        wb = sems[10:12]
        c = lax.axis_index("c")
        s = lax.axis_index("s")
        wid = s * _NC + c

        def base(i):
            return wid * (nb * block) + i * block

        writes = [None, None]
        cp_ia = pltpu.async_copy(src_hbm.at[pl.ds(base(0), block)],
                                 isv.at[0], ia[0])
        cp_ib = pltpu.async_copy(dst_hbm.at[pl.ds(base(0), block)],
                                 idv.at[0], ib[0])
        idx_cp = [(cp_ia, cp_ib), None]
        for i in range(nb):
            sl = i % 2
            nsl = (i + 1) % 2
            idx_cp[sl][0].wait()
            idx_cp[sl][1].wait()
            if writes[sl] is not None:
                writes[sl][0].wait()
                writes[sl][1].wait()
            g1 = pltpu.async_copy(tbl_hbm.at[isv.at[sl]], ra_v.at[sl], ga[sl])
            g2 = pltpu.async_copy(tbl_hbm.at[idv.at[sl]], rb_v.at[sl], gb[sl])
            if i + 1 < nb:
                idx_cp[nsl] = (
                    pltpu.async_copy(src_hbm.at[pl.ds(base(i + 1), block)],
                                     isv.at[nsl], ia[nsl]),
                    pltpu.async_copy(dst_hbm.at[pl.ds(base(i + 1), block)],
                                     idv.at[nsl], ib[nsl]),
                )
            g1.wait()
            g2.wait()
            writes[sl] = (
                pltpu.async_copy(ra_v.at[sl],
                                 xs_hbm.at[pl.ds(base(i), block)], wa[sl]),
                pltpu.async_copy(rb_v.at[sl],
                                 xd_hbm.at[pl.ds(base(i), block)], wb[sl]),
            )
        for w in writes:
            if w is not None:
                w[0].wait()
                w[1].wait()

    return k(tbl, src, dst)


def _sc_cell_reduce(rows, idx, zeros, negs, n_pad, block):
    E = rows.shape[0]
    NB = E // block
    seg = n_pad // _NW
    ngrp = block // 16

    @functools.partial(
        pl.kernel, mesh=_sc_mesh(), compiler_params=_SC_PARAMS,
        out_type=[
            jax.ShapeDtypeStruct((_NW, seg, 16), F32),
            jax.ShapeDtypeStruct((_NW, seg, 16), F32),
        ],
        scratch_types=[
            pltpu.VMEM((block,), jnp.int32),
            pltpu.VMEM((block, 16), F32),
            pltpu.VMEM((seg, 16), F32),
            pltpu.VMEM((seg, 16), F32),
        ],
    )
    def k(rows_hbm, idx_hbm, zeros_hbm, negs_hbm, sum_hbm, max_hbm,
          idx_v, rows_v, acc_sum, acc_max):
        c = lax.axis_index("c")
        s = lax.axis_index("s")
        wid = s * _NC + c
        lo = wid * seg
        hi = lo + seg
        pltpu.sync_copy(zeros_hbm, acc_sum)
        pltpu.sync_copy(negs_hbm, acc_max)

        def blk(j, _):
            base = j * block
            pltpu.sync_copy(idx_hbm.at[pl.ds(base, block)], idx_v)
            pltpu.sync_copy(rows_hbm.at[pl.ds(base, block)], rows_v)

            def grp(t, _):
                iv = idx_v[pl.ds(t * 16, 16)]
                hit = jnp.any((iv >= lo) & (iv < hi))

                @pl.when(hit)
                def _():
                    for e in range(16):
                        ii = iv[e]

                        @pl.when((ii >= lo) & (ii < hi))
                        def _():
                            r = ii - lo
                            row = rows_v[t * 16 + e]
                            acc_sum[r] = acc_sum[r] + row
                            acc_max[r] = jnp.maximum(acc_max[r], row)

                return 0

            lax.fori_loop(0, ngrp, grp, 0)
            return 0

        lax.fori_loop(0, NB, blk, 0)
        pltpu.sync_copy(acc_sum, sum_hbm.at[wid])
        pltpu.sync_copy(acc_max, max_hbm.at[wid])

    sm, mx = k(rows, idx, zeros, negs)
    return sm.reshape(n_pad, 16), mx.reshape(n_pad, 16)


def _net_mlp_body(xa_ref, xb_ref, wa_ref, wb_ref, b1_ref,
                  w2_ref, b2_ref, w3_ref, b3_ref, w4_ref, b4_ref,
                  w5_ref, b5_ref, out_ref):
    xa = xa_ref[...]
    xb = xb_ref[...]
    h = _leaky(jnp.dot(xa, wa_ref[...], preferred_element_type=F32)
               + jnp.dot(xb, wb_ref[...], preferred_element_type=F32)
               + b1_ref[...])
    h = _leaky(jnp.dot(h, w2_ref[...], preferred_element_type=F32) + b2_ref[...])
    h = _leaky(jnp.dot(h, w3_ref[...], preferred_element_type=F32) + b3_ref[...])
    h = _leaky(jnp.dot(h, w4_ref[...], preferred_element_type=F32) + b4_ref[...])
    out_ref[...] = jnp.dot(h, w5_ref[...], preferred_element_type=F32) + b5_ref[...]


def _diag4(w):
    return block_diag(w, w, w, w)


def _tile4(b):
    return jnp.tile(b, 4)[None]


def _run_net_mlp(xa, xb, ps):
    E = xa.shape[0]
    Ep = E // 4
    T = 1000 if Ep % 1000 == 0 else Ep
    (w1, b1), (w2, b2), (w3, b3), (w4, b4), (w5, b5) = ps
    wa = _diag4(_pad_rows(w1[0:28], 32)).astype(BF16)
    wb = _diag4(jnp.zeros((32, 64), F32).at[12:28].set(w1[28:44])).astype(BF16)
    w5p = _diag4(jnp.pad(w5, ((0, 0), (0, 4))))
    b5p = jnp.concatenate([b5, jnp.ones((1,), F32), jnp.zeros((3,), F32)])
    const = lambda shape: pl.BlockSpec(shape, lambda i: (0, 0))
    out = pl.pallas_call(
        _net_mlp_body,
        grid=(Ep // T,),
        in_specs=[
            pl.BlockSpec((T, 128), lambda i: (i, 0)),
            pl.BlockSpec((T, 128), lambda i: (i, 0)),
            const((128, 256)), const((128, 256)), const((1, 256)),
            const((256, 256)), const((1, 256)),
            const((256, 256)), const((1, 256)),
            const((256, 256)), const((1, 256)),
            const((256, 64)), const((1, 64)),
        ],
        out_specs=pl.BlockSpec((T, 64), lambda i: (i, 0)),
        out_shape=jax.ShapeDtypeStruct((Ep, 64), F32),
    )(xa.reshape(Ep, 128), xb.reshape(Ep, 128),
      wa, wb, _tile4(b1), _diag4(w2), _tile4(b2), _diag4(w3), _tile4(b3),
      _diag4(w4), _tile4(b4), w5p, _tile4(b5p))
    return out.reshape(E, 16)


def _cell1_body(xa_ref, xb_ref, ef_ref,
                wa_ref, wb_ref, b1_ref, w2_ref, b2_ref, w3_ref, b3_ref,
                w4_ref, b4_ref, w5_ref, b5_ref,
                wq_ref, bq_ref, wk_ref, bk_ref, wv_ref, bv_ref,
                qqv_ref, kv_ref, vec_ref):
    i = pl.program_id(0)
    xa = xa_ref[...]
    xb = xb_ref[...]
    h = _leaky(jnp.dot(xa, wa_ref[...], preferred_element_type=F32)
               + jnp.dot(xb, wb_ref[...], preferred_element_type=F32)
               + b1_ref[...])
    h = _leaky(jnp.dot(h, w2_ref[...], preferred_element_type=F32) + b2_ref[...])
    h = _leaky(jnp.dot(h, w3_ref[...], preferred_element_type=F32) + b3_ref[...])
    h = _leaky(jnp.dot(h, w4_ref[...], preferred_element_type=F32) + b4_ref[...])
    Q = (jnp.dot(h, w5_ref[...], preferred_element_type=F32) + b5_ref[...]
         + xa[:, 0:12].astype(F32))
    q = jnp.dot(Q, wq_ref[...], preferred_element_type=F32) + bq_ref[...]
    ef = ef_ref[...]
    k = jnp.dot(ef[:, 0:EF_KEY_], wk_ref[...], preferred_element_type=F32) + bk_ref[...]
    v = jnp.dot(ef[:, EF_KEY_:], wv_ref[...], preferred_element_type=F32) + bv_ref[...]
    T = q.shape[0]
    qqv_ref[...] = jnp.concatenate(
        [Q, q, v, jnp.zeros((T, 12), F32)], axis=1)
    kv = lax.dot_general(k, v, (((0,), (0,)), ((), ())),
                         preferred_element_type=F32)
    ksum = jnp.sum(k, axis=0)
    sq = jnp.sum(q * q)
    sk = jnp.sum(k * k)
    row = jnp.concatenate(
        [ksum, jnp.reshape(sq, (1,)), jnp.reshape(sk, (1,)),
         jnp.zeros((114,), F32)])[None]

    @pl.when(i == 0)
    def _():
        kv_ref[...] = jnp.zeros_like(kv_ref)
        vec_ref[...] = jnp.zeros_like(vec_ref)

    kv_ref[...] += kv
    vec_ref[...] += row


def _run_cell1(xa, xb, ef, ps_q, wq, bq, wk, bk, wv, bv):
    E = xa.shape[0]
    T = TE if E % TE == 0 else E
    (w1, b1), (w2, b2), (w3, b3), (w4, b4), (w5, b5) = ps_q
    wa = jnp.concatenate(
        [w1[32:44], w1[0:16], jnp.zeros((4, 64), F32)], axis=0).astype(BF16)
    wb = jnp.zeros((32, 64), F32).at[12:28].set(w1[16:32]).astype(BF16)
    const = lambda shape: pl.BlockSpec(shape, lambda i: (0, 0))
    return pl.pallas_call(
        _cell1_body,
        grid=(E // T,),
        in_specs=[
            pl.BlockSpec((T, 32), lambda i: (i, 0)),
            pl.BlockSpec((T, 32), lambda i: (i, 0)),
            pl.BlockSpec((T, 512), lambda i: (i, 0)),
            const((32, 64)), const((32, 64)), const((1, 64)),
            const((64, 64)), const((1, 64)),
            const((64, 64)), const((1, 64)),
            const((64, 64)), const((1, 64)),
            const((64, 12)), const((1, 12)),
            const((12, 12)), const((1, 12)),
            const((120, 12)), const((1, 12)),
            const((392, 12)), const((1, 12)),
        ],
        out_specs=[
            pl.BlockSpec((T, 48), lambda i: (i, 0)),
            pl.BlockSpec((12, 12), lambda i: (0, 0)),
            pl.BlockSpec((1, 128), lambda i: (0, 0)),
        ],
        out_shape=[
            jax.ShapeDtypeStruct((E, 48), F32),
            jax.ShapeDtypeStruct((12, 12), F32),
            jax.ShapeDtypeStruct((1, 128), F32),
        ],
    )(xa, xb, ef, wa, wb, b1[None], w2, b2[None], w3, b3[None], w4, b4[None],
      w5, b5[None], wq, bq[None], wk, bk[None], wv, bv[None])


def _cell2_body(qqv_ref, xa_ref, xb_ref, kv_ref, vec_ref,
                wa_ref, wb_ref, wy_ref, b1_ref, w2_ref, b2_ref,
                w3_ref, b3_ref, w4_ref, b4_ref, w5_ref, b5_ref,
                g_ref, bb_ref, nedges_ref,
                mji_ref, efce_ref):
    qqv = qqv_ref[...]
    Q = qqv[:, 0:12]
    q = qqv[:, 12:24]
    v = qqv[:, 24:36]
    vec = vec_ref[...]
    ksum = vec[0, 0:12]
    sq = vec[0, 12]
    sk = vec[0, 13]
    NE = nedges_ref[0, 0]
    inv_nq = lax.rsqrt(sq)
    inv_nk = lax.rsqrt(sk)
    qn = q * inv_nq
    kv = kv_ref[...] * inv_nk
    ks = ksum * inv_nk
    att_num = jnp.dot(qn, kv, preferred_element_type=F32) + NE * v
    att_den = jnp.dot(qn, ks[:, None], preferred_element_type=F32) + NE
    O = att_num / att_den
    x = Q + O
    mu = jnp.mean(x, axis=1, keepdims=True)
    var = jnp.mean((x - mu) ** 2, axis=1, keepdims=True)
    y = (x - mu) * lax.rsqrt(var + 1e-5) * g_ref[...] + bb_ref[...]
    xa = xa_ref[...]
    xb = xb_ref[...]
    h = _leaky(jnp.dot(xa, wa_ref[...], preferred_element_type=F32)
               + jnp.dot(xb, wb_ref[...], preferred_element_type=F32)
               + jnp.dot(y, wy_ref[...], preferred_element_type=F32)
               + b1_ref[...])
    h = _leaky(jnp.dot(h, w2_ref[...], preferred_element_type=F32) + b2_ref[...])
    h = _leaky(jnp.dot(h, w3_ref[...], preferred_element_type=F32) + b3_ref[...])
    h = _leaky(jnp.dot(h, w4_ref[...], preferred_element_type=F32) + b4_ref[...])
    xm = jnp.dot(h, w5_ref[...], preferred_element_type=F32) + b5_ref[...]
    T = xm.shape[0]
    mji = xm[:, 0:12] + xa[:, 0:12].astype(F32)
    mji_ref[...] = jnp.concatenate(
        [mji, jnp.ones((T, 1), F32), jnp.zeros((T, 3), F32)], axis=1)
    efce_ref[...] = xm[:, 12:16]


def _run_cell2(qqv, xa, xb, kv, vec, ps_m, ln_g, ln_b):
    E = xa.shape[0]
    T = TE if E % TE == 0 else E
    (w1, b1), (w2, b2), (w3, b3), (w4, b4), (w5, b5) = ps_m
    wa = jnp.concatenate(
        [w1[32:44], w1[0:16], jnp.zeros((4, 64), F32)], axis=0).astype(BF16)
    wb = jnp.zeros((32, 64), F32).at[12:28].set(w1[16:32]).astype(BF16)
    wy = w1[44:56]
    nedges = jnp.full((1, 1), float(E), F32)
    const = lambda shape: pl.BlockSpec(shape, lambda i: (0, 0))
    return pl.pallas_call(
        _cell2_body,
        grid=(E // T,),
        in_specs=[
            pl.BlockSpec((T, 48), lambda i: (i, 0)),
            pl.BlockSpec((T, 32), lambda i: (i, 0)),
            pl.BlockSpec((T, 32), lambda i: (i, 0)),
            const((12, 12)), const((1, 128)),
            const((32, 64)), const((32, 64)), const((12, 64)), const((1, 64)),
            const((64, 64)), const((1, 64)),
            const((64, 64)), const((1, 64)),
            const((64, 64)), const((1, 64)),
            const((64, 16)), const((1, 16)),
            const((1, 12)), const((1, 12)), const((1, 1)),
        ],
        out_specs=[
            pl.BlockSpec((T, 16), lambda i: (i, 0)),
            pl.BlockSpec((T, 4), lambda i: (i, 0)),
        ],
        out_shape=[
            jax.ShapeDtypeStruct((E, 16), F32),
            jax.ShapeDtypeStruct((E, 4), F32),
        ],
    )(qqv, xa, xb, kv, vec, wa, wb, wy, b1[None], w2, b2[None], w3, b3[None],
      w4, b4[None], w5, b5[None], ln_g[None], ln_b[None], nedges)


def _final_body(csum_ref, cmax_ref, nsum0_ref, nsum1_ref, pip0_ref, pip1_ref,
                lastnf_ref, nf_ref,
                wmean_ref, wmax_ref, wnf_ref, b1_ref, w2_ref, b2_ref,
                w3_ref, b3_ref, w4_ref, b4_ref, w5_ref, b5_ref,
                out_ref):
    csum = csum_ref[...]
    cnt = csum[:, 12:13]
    cmask = cnt > 0
    mean = csum[:, 0:12] / jnp.maximum(cnt, 1.0)
    mx = jnp.where(cmask, cmax_ref[:, 0:12], 0.0)
    nf = nf_ref[...]
    h = _leaky(jnp.dot(mean, wmean_ref[...], preferred_element_type=F32)
               + jnp.dot(mx, wmax_ref[...], preferred_element_type=F32)
               + jnp.dot(nf, wnf_ref[...], preferred_element_type=F32)
               + b1_ref[...])
    h = _leaky(jnp.dot(h, w2_ref[...], preferred_element_type=F32) + b2_ref[...])
    h = _leaky(jnp.dot(h, w3_ref[...], preferred_element_type=F32) + b3_ref[...])
    h = _leaky(jnp.dot(h, w4_ref[...], preferred_element_type=F32) + b4_ref[...])
    red = jnp.dot(h, w5_ref[...], preferred_element_type=F32) + b5_ref[...]
    nsum = nsum0_ref[0] + nsum1_ref[0]
    nmask = nsum[:, 12:13] > 0
    pmask = (pip0_ref[0, :, 12:13] + pip1_ref[0, :, 12:13]) > 0
    base = jnp.where(pmask, lastnf_ref[...], 0.0)
    out_ref[...] = jnp.where(cmask, red,
                             jnp.where(nmask, nsum[:, 0:12], base))


def _run_final(csum, cmax, nsum_parts, pi_parts, last_nf, nf, ps_a):
    N = nf.shape[0]
    T = TN if N % TN == 0 else N
    (w1, b1), (w2, b2), (w3, b3), (w4, b4), (w5, b5) = ps_a
    wmean = w1[0:12]
    wmax = w1[12:24]
    wnf = w1[24:40]
    const = lambda shape: pl.BlockSpec(shape, lambda i: (0, 0))
    return pl.pallas_call(
        _final_body,
        grid=(N // T,),
        in_specs=[
            pl.BlockSpec((T, 16), lambda i: (i, 0)),
            pl.BlockSpec((T, 16), lambda i: (i, 0)),
            pl.BlockSpec((1, T, 16), lambda i: (0, i, 0)),
            pl.BlockSpec((1, T, 16), lambda i: (1, i, 0)),
            pl.BlockSpec((1, T, 16), lambda i: (0, i, 0)),
            pl.BlockSpec((1, T, 16), lambda i: (1, i, 0)),
            pl.BlockSpec((T, 12), lambda i: (i, 0)),
            pl.BlockSpec((T, 16), lambda i: (i, 0)),
            const((12, 64)), const((12, 64)), const((16, 64)), const((1, 64)),
            const((64, 64)), const((1, 64)),
            const((64, 64)), const((1, 64)),
            const((64, 64)), const((1, 64)),
            const((64, 12)), const((1, 12)),
        ],
        out_specs=pl.BlockSpec((T, 12), lambda i: (i, 0)),
        out_shape=jax.ShapeDtypeStruct((N, 12), F32),
    )(csum, cmax, nsum_parts, nsum_parts, pi_parts, pi_parts, last_nf, nf,
      wmean, wmax, wnf, b1[None],
      w2, b2[None], w3, b3[None], w4, b4[None], w5, b5[None])


def kernel(nf, n_ats, n_slew_log, n_net_delays_log, ef, params,
           edge_index_net, edge_index_cell, pi_nodes):
    N = nf.shape[0]
    last_nf = jnp.concatenate([n_ats, n_slew_log, n_net_delays_log], axis=1)
    tbl = jnp.concatenate(
        [last_nf, nf, jnp.zeros((N, 4), F32)], axis=1).astype(BF16)
    zeros16 = jnp.zeros((N, 16), F32)

    src = edge_index_net[0].astype(jnp.int32)
    dst = edge_index_net[1].astype(jnp.int32)
    xa_n, xb_n = _sc_edge_gather_pipe(tbl, src, dst, 1000)
    efn = _run_net_mlp(xa_n, xb_n, params['netprop'])
    net_parts = _sc_scatter_add_pipe(efn, dst, zeros16, N, 1000)

    pi32 = pi_nodes.astype(jnp.int32)
    pi_parts = _sc_scatter_add(
        jnp.ones((pi32.shape[0], 16), F32), pi32, zeros16, N, pi32.shape[0])

    s = edge_index_cell[0].astype(jnp.int32)
    d = edge_index_cell[1].astype(jnp.int32)
    xa_c, xb_c = _sc_edge_gather(tbl, s, d, 1000)
    wq, bq = params['Wq']
    wk, bk = params['Wk']
    wv, bv = params['Wv']
    qqv, kv, vec = _run_cell1(xa_c, xb_c, ef, params['q'],
                              wq, bq, wk, bk, wv, bv)
    mji, efce = _run_cell2(qqv, xa_c, xb_c, kv, vec, params['m'],
                           params['ln_g'], params['ln_b'])
    NPAD = 50048
    seg = NPAD // 32
    csum_p, cmax_p = _sc_cell_reduce(
        mji, d, jnp.zeros((seg, 16), F32), jnp.full((seg, 16), -1e30, F32),
        NPAD, 800)
    cell_sum = csum_p[:N]
    cell_max = cmax_p[:N]

    new_nf = _run_final(cell_sum, cell_max, net_parts, pi_parts, last_nf, nf,
                        params['a'])
    return (new_nf, efce)


# speedup vs baseline: 1.0132x; 1.0132x over previous
import functools

import jax
import jax.numpy as jnp
from jax import lax
from jax.scipy.linalg import block_diag
from jax.experimental import pallas as pl
from jax.experimental.pallas import tpu as pltpu
from jax.experimental.pallas import tpu_sc as plsc

F32 = jnp.float32
BF16 = jnp.bfloat16

IN_NF_ = 16
OUT_NF_ = 12
EF_KEY_ = 120
OUT_CEF_ = 4

TE = 1000
TN = 1000


def _leaky(x):
    return jnp.where(x > 0, x, 0.2 * x)


def _pad_rows(w, rows):
    return jnp.pad(w, ((0, rows - w.shape[0]), (0, 0)))


_NC, _NS, _NW = 2, 16, 32


def _sc_mesh():
    return plsc.VectorSubcoreMesh(core_axis_name="c", subcore_axis_name="s")


_SC_PARAMS = pltpu.CompilerParams(use_tc_tiling_on_sc=False,
                                  needs_layout_passes=False)


def _sc_scatter_add(rows, idx, zeros, n_nodes, block):
    E = rows.shape[0]
    NB = E // block
    nb_per_w = (NB + _NW - 1) // _NW
    rows_seg = n_nodes // _NS

    @functools.partial(
        pl.kernel, mesh=_sc_mesh(), compiler_params=_SC_PARAMS,
        out_type=jax.ShapeDtypeStruct((_NC, _NS, rows_seg, 16), F32),
        scratch_types=[
            pltpu.VMEM((block,), jnp.int32),
            pltpu.VMEM((block, 16), F32),
            pltpu.VMEM_SHARED((n_nodes, 16), F32),
        ],
    )
    def k(rows_hbm, idx_hbm, zeros_hbm, out_hbm, idx_v, rows_v, acc_s):
        c = lax.axis_index("c")
        s = lax.axis_index("s")
        wid = s * _NC + c
        pltpu.sync_copy(zeros_hbm.at[s],
                        acc_s.at[pl.ds(s * rows_seg, rows_seg)])
        plsc.subcore_barrier()

        def body(i, _):
            j = i * _NW + wid

            @pl.when(j < NB)
            def _():
                base = j * block
                pltpu.sync_copy(idx_hbm.at[pl.ds(base, block)], idx_v)
                pltpu.sync_copy(rows_hbm.at[pl.ds(base, block)], rows_v)
                pltpu.sync_copy(rows_v, acc_s.at[idx_v], add=True)

            return 0

        lax.fori_loop(0, nb_per_w, body, 0)
        plsc.subcore_barrier()
        pltpu.sync_copy(acc_s.at[pl.ds(s * rows_seg, rows_seg)],
                        out_hbm.at[c, s])

    out = k(rows, idx, zeros.reshape(_NS, rows_seg, 16))
    return out.reshape(_NC, n_nodes, 16)


def _sc_scatter_add_pipe(rows, idx, zeros, n_nodes, block):
    E = rows.shape[0]
    nb = E // (_NW * block)
    rows_seg = n_nodes // _NS

    @functools.partial(
        pl.kernel, mesh=_sc_mesh(), compiler_params=_SC_PARAMS,
        out_type=jax.ShapeDtypeStruct((_NC, _NS, rows_seg, 16), F32),
        scratch_types=[
            pltpu.VMEM((2, block), jnp.int32),
            pltpu.VMEM((2, block, 16), F32),
            pltpu.VMEM_SHARED((n_nodes, 16), F32),
        ] + [pltpu.SemaphoreType.DMA] * 6,
    )
    def k(rows_hbm, idx_hbm, zeros_hbm, out_hbm, idx_v, rows_v, acc_s, *sems):
        si = sems[0:2]
        sr = sems[2:4]
        ss = sems[4:6]
        c = lax.axis_index("c")
        s = lax.axis_index("s")
        wid = s * _NC + c
        pltpu.sync_copy(zeros_hbm.at[s],
                        acc_s.at[pl.ds(s * rows_seg, rows_seg)])
        plsc.subcore_barrier()

        def base(i):
            return wid * (nb * block) + i * block

        loads = [None, None]
        scat = [None, None]
        loads[0] = (
            pltpu.async_copy(idx_hbm.at[pl.ds(base(0), block)],
                             idx_v.at[0], si[0]),
            pltpu.async_copy(rows_hbm.at[pl.ds(base(0), block)],
                             rows_v.at[0], sr[0]),
        )
        for i in range(nb):
            sl = i % 2
            nsl = (i + 1) % 2
            loads[sl][0].wait()
            loads[sl][1].wait()
            scat[sl] = pltpu.async_copy(rows_v.at[sl],
                                        acc_s.at[idx_v.at[sl]], ss[sl],
                                        add=True)
            if i + 1 < nb:
                if scat[nsl] is not None:
                    scat[nsl].wait()
                loads[nsl] = (
                    pltpu.async_copy(idx_hbm.at[pl.ds(base(i + 1), block)],
                                     idx_v.at[nsl], si[nsl]),
                    pltpu.async_copy(rows_hbm.at[pl.ds(base(i + 1), block)],
                                     rows_v.at[nsl], sr[nsl]),
                )
        scat[(nb - 1) % 2].wait()
        plsc.subcore_barrier()
        pltpu.sync_copy(acc_s.at[pl.ds(s * rows_seg, rows_seg)],
                        out_hbm.at[c, s])

    out = k(rows, idx, zeros.reshape(_NS, rows_seg, 16))
    return out.reshape(_NC, n_nodes, 16)


def _sc_edge_gather(tbl, src, dst, block):
    E = src.shape[0]
    NB = E // block
    nb_per_w = (NB + _NW - 1) // _NW

    @functools.partial(
        pl.kernel, mesh=_sc_mesh(), compiler_params=_SC_PARAMS,
        out_type=[
            jax.ShapeDtypeStruct((E, 32), BF16),
            jax.ShapeDtypeStruct((E, 32), BF16),
        ],
        scratch_types=[
            pltpu.VMEM((block,), jnp.int32),
            pltpu.VMEM((block, 32), BF16),
            pltpu.VMEM((block, 32), BF16),
        ],
    )
    def k(tbl_hbm, src_hbm, dst_hbm, xs_hbm, xd_hbm, idx_v, ra_v, rb_v):
        c = lax.axis_index("c")
        s = lax.axis_index("s")
        wid = s * _NC + c

        def body(i, _):
            j = i * _NW + wid

            @pl.when(j < NB)
            def _():
                base = j * block
                pltpu.sync_copy(src_hbm.at[pl.ds(base, block)], idx_v)
                pltpu.sync_copy(tbl_hbm.at[idx_v], ra_v)
                pltpu.sync_copy(ra_v, xs_hbm.at[pl.ds(base, block)])
                pltpu.sync_copy(dst_hbm.at[pl.ds(base, block)], idx_v)
                pltpu.sync_copy(tbl_hbm.at[idx_v], rb_v)
                pltpu.sync_copy(rb_v, xd_hbm.at[pl.ds(base, block)])

            return 0

        lax.fori_loop(0, nb_per_w, body, 0)

    return k(tbl, src, dst)


def _sc_edge_gather_pipe(tbl, src, dst, block):
    E = src.shape[0]
    nb = E // (_NW * block)

    @functools.partial(
        pl.kernel, mesh=_sc_mesh(), compiler_params=_SC_PARAMS,
        out_type=[
            jax.ShapeDtypeStruct((E, 32), BF16),
            jax.ShapeDtypeStruct((E, 32), BF16),
        ],
        scratch_types=[
            pltpu.VMEM((2, block), jnp.int32),
            pltpu.VMEM((2, block), jnp.int32),
            pltpu.VMEM((2, block, 32), BF16),
            pltpu.VMEM((2, block, 32), BF16),
        ] + [pltpu.SemaphoreType.DMA] * 12,
    )
    def k(tbl_hbm, src_hbm, dst_hbm, xs_hbm, xd_hbm,
          isv, idv, ra_v, rb_v, *sems):
        ia = sems[0:2]
        ib = sems[2:4]
        ga = sems[4:6]
        gb = sems[6:8]
        wa = sems[8:10]
        wb = sems[10:12]
        c = lax.axis_index("c")
        s = lax.axis_index("s")
        wid = s * _NC + c

        def base(i):
            return wid * (nb * block) + i * block

        writes = [None, None]
        cp_ia = pltpu.async_copy(src_hbm.at[pl.ds(base(0), block)],
                                 isv.at[0], ia[0])
        cp_ib = pltpu.async_copy(dst_hbm.at[pl.ds(base(0), block)],
                                 idv.at[0], ib[0])
        idx_cp = [(cp_ia, cp_ib), None]
        for i in range(nb):
            sl = i % 2
            nsl = (i + 1) % 2
            idx_cp[sl][0].wait()
            idx_cp[sl][1].wait()
            if writes[sl] is not None:
                writes[sl][0].wait()
                writes[sl][1].wait()
            g1 = pltpu.async_copy(tbl_hbm.at[isv.at[sl]], ra_v.at[sl], ga[sl])
            g2 = pltpu.async_copy(tbl_hbm.at[idv.at[sl]], rb_v.at[sl], gb[sl])
            if i + 1 < nb:
                idx_cp[nsl] = (
                    pltpu.async_copy(src_hbm.at[pl.ds(base(i + 1), block)],
                                     isv.at[nsl], ia[nsl]),
                    pltpu.async_copy(dst_hbm.at[pl.ds(base(i + 1), block)],
                                     idv.at[nsl], ib[nsl]),
                )
            g1.wait()
            g2.wait()
            writes[sl] = (
                pltpu.async_copy(ra_v.at[sl],
                                 xs_hbm.at[pl.ds(base(i), block)], wa[sl]),
                pltpu.async_copy(rb_v.at[sl],
                                 xd_hbm.at[pl.ds(base(i), block)], wb[sl]),
            )
        for w in writes:
            if w is not None:
                w[0].wait()
                w[1].wait()

    return k(tbl, src, dst)


def _sc_cell_reduce(rows, idx, zeros, negs, n_pad, block):
    E = rows.shape[0]
    NB = E // block
    seg = n_pad // _NW
    ngrp = block // 16

    @functools.partial(
        pl.kernel, mesh=_sc_mesh(), compiler_params=_SC_PARAMS,
        out_type=[
            jax.ShapeDtypeStruct((_NW, seg, 16), F32),
            jax.ShapeDtypeStruct((_NW, seg, 16), F32),
        ],
        scratch_types=[
            pltpu.VMEM((block,), jnp.int32),
            pltpu.VMEM((block, 16), F32),
            pltpu.VMEM((seg, 16), F32),
            pltpu.VMEM((seg, 16), F32),
        ],
    )
    def k(rows_hbm, idx_hbm, zeros_hbm, negs_hbm, sum_hbm, max_hbm,
          idx_v, rows_v, acc_sum, acc_max):
        c = lax.axis_index("c")
        s = lax.axis_index("s")
        wid = s * _NC + c
        lo = wid * seg
        hi = lo + seg
        pltpu.sync_copy(zeros_hbm, acc_sum)
        pltpu.sync_copy(negs_hbm, acc_max)

        def blk(j, _):
            base = j * block
            pltpu.sync_copy(idx_hbm.at[pl.ds(base, block)], idx_v)
            pltpu.sync_copy(rows_hbm.at[pl.ds(base, block)], rows_v)

            def grp(t, _):
                iv = idx_v[pl.ds(t * 16, 16)]
                hit = jnp.any((iv >= lo) & (iv < hi))

                @pl.when(hit)
                def _():
                    for e in range(16):
                        ii = iv[e]

                        @pl.when((ii >= lo) & (ii < hi))
                        def _():
                            r = ii - lo
                            row = rows_v[t * 16 + e]
                            acc_sum[r] = acc_sum[r] + row
                            acc_max[r] = jnp.maximum(acc_max[r], row)

                return 0

            lax.fori_loop(0, ngrp, grp, 0)
            return 0

        lax.fori_loop(0, NB, blk, 0)
        pltpu.sync_copy(acc_sum, sum_hbm.at[wid])
        pltpu.sync_copy(acc_max, max_hbm.at[wid])

    sm, mx = k(rows, idx, zeros, negs)
    return sm.reshape(n_pad, 16), mx.reshape(n_pad, 16)


def _net_mlp_body(xa_ref, xb_ref, wa_ref, wb_ref, b1_ref,
                  w2_ref, b2_ref, w3_ref, b3_ref, w4_ref, b4_ref,
                  w5_ref, b5_ref, out_ref):
    xa = xa_ref[...].astype(F32)
    xb = xb_ref[...].astype(F32)
    h = _leaky(jnp.dot(xa, wa_ref[...], preferred_element_type=F32)
               + jnp.dot(xb, wb_ref[...], preferred_element_type=F32)
               + b1_ref[...])
    h = _leaky(jnp.dot(h, w2_ref[...], preferred_element_type=F32) + b2_ref[...])
    h = _leaky(jnp.dot(h, w3_ref[...], preferred_element_type=F32) + b3_ref[...])
    h = _leaky(jnp.dot(h, w4_ref[...], preferred_element_type=F32) + b4_ref[...])
    out_ref[...] = jnp.dot(h, w5_ref[...], preferred_element_type=F32) + b5_ref[...]


def _diag4(w):
    return block_diag(w, w, w, w)


def _tile4(b):
    return jnp.tile(b, 4)[None]


def _run_net_mlp(xa, xb, ps):
    E = xa.shape[0]
    Ep = E // 4
    T = 1000 if Ep % 1000 == 0 else Ep
    (w1, b1), (w2, b2), (w3, b3), (w4, b4), (w5, b5) = ps
    wa = _diag4(_pad_rows(w1[0:28], 32))
    wb = _diag4(jnp.zeros((32, 64), F32).at[12:28].set(w1[28:44]))
    w5p = _diag4(jnp.pad(w5, ((0, 0), (0, 4))))
    b5p = jnp.concatenate([b5, jnp.ones((1,), F32), jnp.zeros((3,), F32)])
    const = lambda shape: pl.BlockSpec(shape, lambda i: (0, 0))
    out = pl.pallas_call(
        _net_mlp_body,
        grid=(Ep // T,),
        in_specs=[
            pl.BlockSpec((T, 128), lambda i: (i, 0)),
            pl.BlockSpec((T, 128), lambda i: (i, 0)),
            const((128, 256)), const((128, 256)), const((1, 256)),
            const((256, 256)), const((1, 256)),
            const((256, 256)), const((1, 256)),
            const((256, 256)), const((1, 256)),
            const((256, 64)), const((1, 64)),
        ],
        out_specs=pl.BlockSpec((T, 64), lambda i: (i, 0)),
        out_shape=jax.ShapeDtypeStruct((Ep, 64), F32),
    )(xa.reshape(Ep, 128), xb.reshape(Ep, 128),
      wa, wb, _tile4(b1), _diag4(w2), _tile4(b2), _diag4(w3), _tile4(b3),
      _diag4(w4), _tile4(b4), w5p, _tile4(b5p))
    return out.reshape(E, 16)


def _cell1_body(xa_ref, xb_ref, ef_ref,
                wa_ref, wb_ref, b1_ref, w2_ref, b2_ref, w3_ref, b3_ref,
                w4_ref, b4_ref, w5_ref, b5_ref,
                wq_ref, bq_ref, wk_ref, bk_ref, wv_ref, bv_ref,
                qqv_ref, kv_ref, vec_ref):
    i = pl.program_id(0)
    xa = xa_ref[...].astype(F32)
    xb = xb_ref[...].astype(F32)
    h = _leaky(jnp.dot(xa, wa_ref[...], preferred_element_type=F32)
               + jnp.dot(xb, wb_ref[...], preferred_element_type=F32)
               + b1_ref[...])
    h = _leaky(jnp.dot(h, w2_ref[...], preferred_element_type=F32) + b2_ref[...])
    h = _leaky(jnp.dot(h, w3_ref[...], preferred_element_type=F32) + b3_ref[...])
    h = _leaky(jnp.dot(h, w4_ref[...], preferred_element_type=F32) + b4_ref[...])
    Q = (jnp.dot(h, w5_ref[...], preferred_element_type=F32) + b5_ref[...]
         + xa[:, 0:12].astype(F32))
    q = jnp.dot(Q, wq_ref[...], preferred_element_type=F32) + bq_ref[...]
    ef = ef_ref[...]
    k = jnp.dot(ef[:, 0:EF_KEY_], wk_ref[...], preferred_element_type=F32) + bk_ref[...]
    v = jnp.dot(ef[:, EF_KEY_:], wv_ref[...], preferred_element_type=F32) + bv_ref[...]
    T = q.shape[0]
    qqv_ref[...] = jnp.concatenate(
        [Q, q, v, jnp.zeros((T, 12), F32)], axis=1)
    kv = lax.dot_general(k, v, (((0,), (0,)), ((), ())),
                         preferred_element_type=F32)
    ksum = jnp.sum(k, axis=0)
    sq = jnp.sum(q * q)
    sk = jnp.sum(k * k)
    row = jnp.concatenate(
        [ksum, jnp.reshape(sq, (1,)), jnp.reshape(sk, (1,)),
         jnp.zeros((114,), F32)])[None]

    @pl.when(i == 0)
    def _():
        kv_ref[...] = jnp.zeros_like(kv_ref)
        vec_ref[...] = jnp.zeros_like(vec_ref)

    kv_ref[...] += kv
    vec_ref[...] += row


def _run_cell1(xa, xb, ef, ps_q, wq, bq, wk, bk, wv, bv):
    E = xa.shape[0]
    T = TE if E % TE == 0 else E
    (w1, b1), (w2, b2), (w3, b3), (w4, b4), (w5, b5) = ps_q
    wa = jnp.concatenate(
        [w1[32:44], w1[0:16], jnp.zeros((4, 64), F32)], axis=0)
    wb = jnp.zeros((32, 64), F32).at[12:28].set(w1[16:32])
    const = lambda shape: pl.BlockSpec(shape, lambda i: (0, 0))
    return pl.pallas_call(
        _cell1_body,
        grid=(E // T,),
        in_specs=[
            pl.BlockSpec((T, 32), lambda i: (i, 0)),
            pl.BlockSpec((T, 32), lambda i: (i, 0)),
            pl.BlockSpec((T, 512), lambda i: (i, 0)),
            const((32, 64)), const((32, 64)), const((1, 64)),
            const((64, 64)), const((1, 64)),
            const((64, 64)), const((1, 64)),
            const((64, 64)), const((1, 64)),
            const((64, 12)), const((1, 12)),
            const((12, 12)), const((1, 12)),
            const((120, 12)), const((1, 12)),
            const((392, 12)), const((1, 12)),
        ],
        out_specs=[
            pl.BlockSpec((T, 48), lambda i: (i, 0)),
            pl.BlockSpec((12, 12), lambda i: (0, 0)),
            pl.BlockSpec((1, 128), lambda i: (0, 0)),
        ],
        out_shape=[
            jax.ShapeDtypeStruct((E, 48), F32),
            jax.ShapeDtypeStruct((12, 12), F32),
            jax.ShapeDtypeStruct((1, 128), F32),
        ],
    )(xa, xb, ef, wa, wb, b1[None], w2, b2[None], w3, b3[None], w4, b4[None],
      w5, b5[None], wq, bq[None], wk, bk[None], wv, bv[None])


def _cell2_body(qqv_ref, xa_ref, xb_ref, kv_ref, vec_ref,
                wa_ref, wb_ref, wy_ref, b1_ref, w2_ref, b2_ref,
                w3_ref, b3_ref, w4_ref, b4_ref, w5_ref, b5_ref,
                g_ref, bb_ref, nedges_ref,
                mji_ref, efce_ref):
    qqv = qqv_ref[...]
    Q = qqv[:, 0:12]
    q = qqv[:, 12:24]
    v = qqv[:, 24:36]
    vec = vec_ref[...]
    ksum = vec[0, 0:12]
    sq = vec[0, 12]
    sk = vec[0, 13]
    NE = nedges_ref[0, 0]
    inv_nq = lax.rsqrt(sq)
    inv_nk = lax.rsqrt(sk)
    qn = q * inv_nq
    kv = kv_ref[...] * inv_nk
    ks = ksum * inv_nk
    att_num = jnp.dot(qn, kv, preferred_element_type=F32) + NE * v
    att_den = jnp.dot(qn, ks[:, None], preferred_element_type=F32) + NE
    O = att_num / att_den
    x = Q + O
    mu = jnp.mean(x, axis=1, keepdims=True)
    var = jnp.mean((x - mu) ** 2, axis=1, keepdims=True)
    y = (x - mu) * lax.rsqrt(var + 1e-5) * g_ref[...] + bb_ref[...]
    xa = xa_ref[...].astype(F32)
    xb = xb_ref[...].astype(F32)
    h = _leaky(jnp.dot(xa, wa_ref[...], preferred_element_type=F32)
               + jnp.dot(xb, wb_ref[...], preferred_element_type=F32)
               + jnp.dot(y, wy_ref[...], preferred_element_type=F32)
               + b1_ref[...])
    h = _leaky(jnp.dot(h, w2_ref[...], preferred_element_type=F32) + b2_ref[...])
    h = _leaky(jnp.dot(h, w3_ref[...], preferred_element_type=F32) + b3_ref[...])
    h = _leaky(jnp.dot(h, w4_ref[...], preferred_element_type=F32) + b4_ref[...])
    xm = jnp.dot(h, w5_ref[...], preferred_element_type=F32) + b5_ref[...]
    T = xm.shape[0]
    mji = xm[:, 0:12] + xa[:, 0:12].astype(F32)
    mji_ref[...] = jnp.concatenate(
        [mji, jnp.ones((T, 1), F32), jnp.zeros((T, 3), F32)], axis=1)
    efce_ref[...] = xm[:, 12:16]


def _run_cell2(qqv, xa, xb, kv, vec, ps_m, ln_g, ln_b):
    E = xa.shape[0]
    T = TE if E % TE == 0 else E
    (w1, b1), (w2, b2), (w3, b3), (w4, b4), (w5, b5) = ps_m
    wa = jnp.concatenate(
        [w1[32:44], w1[0:16], jnp.zeros((4, 64), F32)], axis=0)
    wb = jnp.zeros((32, 64), F32).at[12:28].set(w1[16:32])
    wy = w1[44:56]
    nedges = jnp.full((1, 1), float(E), F32)
    const = lambda shape: pl.BlockSpec(shape, lambda i: (0, 0))
    return pl.pallas_call(
        _cell2_body,
        grid=(E // T,),
        in_specs=[
            pl.BlockSpec((T, 48), lambda i: (i, 0)),
            pl.BlockSpec((T, 32), lambda i: (i, 0)),
            pl.BlockSpec((T, 32), lambda i: (i, 0)),
            const((12, 12)), const((1, 128)),
            const((32, 64)), const((32, 64)), const((12, 64)), const((1, 64)),
            const((64, 64)), const((1, 64)),
            const((64, 64)), const((1, 64)),
            const((64, 64)), const((1, 64)),
            const((64, 16)), const((1, 16)),
            const((1, 12)), const((1, 12)), const((1, 1)),
        ],
        out_specs=[
            pl.BlockSpec((T, 16), lambda i: (i, 0)),
            pl.BlockSpec((T, 4), lambda i: (i, 0)),
        ],
        out_shape=[
            jax.ShapeDtypeStruct((E, 16), F32),
            jax.ShapeDtypeStruct((E, 4), F32),
        ],
    )(qqv, xa, xb, kv, vec, wa, wb, wy, b1[None], w2, b2[None], w3, b3[None],
      w4, b4[None], w5, b5[None], ln_g[None], ln_b[None], nedges)


def _final_body(csum_ref, cmax_ref, nsum0_ref, nsum1_ref, pip0_ref, pip1_ref,
                lastnf_ref, nf_ref,
                wmean_ref, wmax_ref, wnf_ref, b1_ref, w2_ref, b2_ref,
                w3_ref, b3_ref, w4_ref, b4_ref, w5_ref, b5_ref,
                out_ref):
    csum = csum_ref[...]
    cnt = csum[:, 12:13]
    cmask = cnt > 0
    mean = csum[:, 0:12] / jnp.maximum(cnt, 1.0)
    mx = jnp.where(cmask, cmax_ref[:, 0:12], 0.0)
    nf = nf_ref[...]
    h = _leaky(jnp.dot(mean, wmean_ref[...], preferred_element_type=F32)
               + jnp.dot(mx, wmax_ref[...], preferred_element_type=F32)
               + jnp.dot(nf, wnf_ref[...], preferred_element_type=F32)
               + b1_ref[...])
    h = _leaky(jnp.dot(h, w2_ref[...], preferred_element_type=F32) + b2_ref[...])
    h = _leaky(jnp.dot(h, w3_ref[...], preferred_element_type=F32) + b3_ref[...])
    h = _leaky(jnp.dot(h, w4_ref[...], preferred_element_type=F32) + b4_ref[...])
    red = jnp.dot(h, w5_ref[...], preferred_element_type=F32) + b5_ref[...]
    nsum = nsum0_ref[0] + nsum1_ref[0]
    nmask = nsum[:, 12:13] > 0
    pmask = (pip0_ref[0, :, 12:13] + pip1_ref[0, :, 12:13]) > 0
    base = jnp.where(pmask, lastnf_ref[...], 0.0)
    out_ref[...] = jnp.where(cmask, red,
                             jnp.where(nmask, nsum[:, 0:12], base))


def _run_final(csum, cmax, nsum_parts, pi_parts, last_nf, nf, ps_a):
    N = nf.shape[0]
    T = TN if N % TN == 0 else N
    (w1, b1), (w2, b2), (w3, b3), (w4, b4), (w5, b5) = ps_a
    wmean = w1[0:12]
    wmax = w1[12:24]
    wnf = w1[24:40]
    const = lambda shape: pl.BlockSpec(shape, lambda i: (0, 0))
    return pl.pallas_call(
        _final_body,
        grid=(N // T,),
        in_specs=[
            pl.BlockSpec((T, 16), lambda i: (i, 0)),
            pl.BlockSpec((T, 16), lambda i: (i, 0)),
            pl.BlockSpec((1, T, 16), lambda i: (0, i, 0)),
            pl.BlockSpec((1, T, 16), lambda i: (1, i, 0)),
            pl.BlockSpec((1, T, 16), lambda i: (0, i, 0)),
            pl.BlockSpec((1, T, 16), lambda i: (1, i, 0)),
            pl.BlockSpec((T, 12), lambda i: (i, 0)),
            pl.BlockSpec((T, 16), lambda i: (i, 0)),
            const((12, 64)), const((12, 64)), const((16, 64)), const((1, 64)),
            const((64, 64)), const((1, 64)),
            const((64, 64)), const((1, 64)),
            const((64, 64)), const((1, 64)),
            const((64, 12)), const((1, 12)),
        ],
        out_specs=pl.BlockSpec((T, 12), lambda i: (i, 0)),
        out_shape=jax.ShapeDtypeStruct((N, 12), F32),
    )(csum, cmax, nsum_parts, nsum_parts, pi_parts, pi_parts, last_nf, nf,
      wmean, wmax, wnf, b1[None],
      w2, b2[None], w3, b3[None], w4, b4[None], w5, b5[None])


def kernel(nf, n_ats, n_slew_log, n_net_delays_log, ef, params,
           edge_index_net, edge_index_cell, pi_nodes):
    N = nf.shape[0]
    last_nf = jnp.concatenate([n_ats, n_slew_log, n_net_delays_log], axis=1)
    tbl = jnp.concatenate(
        [last_nf, nf, jnp.zeros((N, 4), F32)], axis=1).astype(BF16)
    zeros16 = jnp.zeros((N, 16), F32)

    src = edge_index_net[0].astype(jnp.int32)
    dst = edge_index_net[1].astype(jnp.int32)
    xa_n, xb_n = _sc_edge_gather_pipe(tbl, src, dst, 1000)
    efn = _run_net_mlp(xa_n, xb_n, params['netprop'])
    net_parts = _sc_scatter_add_pipe(efn, dst, zeros16, N, 1000)

    pi32 = pi_nodes.astype(jnp.int32)
    pi_parts = _sc_scatter_add(
        jnp.ones((pi32.shape[0], 16), F32), pi32, zeros16, N, pi32.shape[0])

    s = edge_index_cell[0].astype(jnp.int32)
    d = edge_index_cell[1].astype(jnp.int32)
    xa_c, xb_c = _sc_edge_gather(tbl, s, d, 1000)
    wq, bq = params['Wq']
    wk, bk = params['Wk']
    wv, bv = params['Wv']
    qqv, kv, vec = _run_cell1(xa_c, xb_c, ef, params['q'],
                              wq, bq, wk, bk, wv, bv)
    mji, efce = _run_cell2(qqv, xa_c, xb_c, kv, vec, params['m'],
                           params['ln_g'], params['ln_b'])
    NPAD = 50048
    seg = NPAD // 32
    csum_p, cmax_p = _sc_cell_reduce(
        mji, d, jnp.zeros((seg, 16), F32), jnp.full((seg, 16), -1e30, F32),
        NPAD, 800)
    cell_sum = csum_p[:N]
    cell_max = cmax_p[:N]

    new_nf = _run_final(cell_sum, cell_max, net_parts, pi_parts, last_nf, nf,
                        params['a'])
    return (new_nf, efce)


# speedup vs baseline: 1.1221x; 1.1075x over previous
import functools

import jax
import jax.numpy as jnp
from jax import lax
from jax.scipy.linalg import block_diag
from jax.experimental import pallas as pl
from jax.experimental.pallas import tpu as pltpu
from jax.experimental.pallas import tpu_sc as plsc

F32 = jnp.float32
BF16 = jnp.bfloat16

IN_NF_ = 16
OUT_NF_ = 12
EF_KEY_ = 120
OUT_CEF_ = 4

TE = 1000
TN = 1000


def _leaky(x):
    return jnp.where(x > 0, x, 0.2 * x)


def _pad_rows(w, rows):
    return jnp.pad(w, ((0, rows - w.shape[0]), (0, 0)))


_NC, _NS, _NW = 2, 16, 32


def _sc_mesh():
    return plsc.VectorSubcoreMesh(core_axis_name="c", subcore_axis_name="s")


_SC_PARAMS = pltpu.CompilerParams(use_tc_tiling_on_sc=False,
                                  needs_layout_passes=False)


def _sc_scatter_add(rows, idx, zeros, n_nodes, block):
    E = rows.shape[0]
    NB = E // block
    nb_per_w = (NB + _NW - 1) // _NW
    rows_seg = n_nodes // _NS

    @functools.partial(
        pl.kernel, mesh=_sc_mesh(), compiler_params=_SC_PARAMS,
        out_type=jax.ShapeDtypeStruct((_NC, _NS, rows_seg, 16), F32),
        scratch_types=[
            pltpu.VMEM((block,), jnp.int32),
            pltpu.VMEM((block, 16), F32),
            pltpu.VMEM_SHARED((n_nodes, 16), F32),
        ],
    )
    def k(rows_hbm, idx_hbm, zeros_hbm, out_hbm, idx_v, rows_v, acc_s):
        c = lax.axis_index("c")
        s = lax.axis_index("s")
        wid = s * _NC + c
        pltpu.sync_copy(zeros_hbm.at[s],
                        acc_s.at[pl.ds(s * rows_seg, rows_seg)])
        plsc.subcore_barrier()

        def body(i, _):
            j = i * _NW + wid

            @pl.when(j < NB)
            def _():
                base = j * block
                pltpu.sync_copy(idx_hbm.at[pl.ds(base, block)], idx_v)
                pltpu.sync_copy(rows_hbm.at[pl.ds(base, block)], rows_v)
                pltpu.sync_copy(rows_v, acc_s.at[idx_v], add=True)

            return 0

        lax.fori_loop(0, nb_per_w, body, 0)
        plsc.subcore_barrier()
        pltpu.sync_copy(acc_s.at[pl.ds(s * rows_seg, rows_seg)],
                        out_hbm.at[c, s])

    out = k(rows, idx, zeros.reshape(_NS, rows_seg, 16))
    return out.reshape(_NC, n_nodes, 16)


def _sc_scatter_add_pipe(rows, idx, zeros, n_nodes, block):
    E = rows.shape[0]
    nb = E // (_NW * block)
    rows_seg = n_nodes // _NS

    @functools.partial(
        pl.kernel, mesh=_sc_mesh(), compiler_params=_SC_PARAMS,
        out_type=jax.ShapeDtypeStruct((_NC, _NS, rows_seg, 16), F32),
        scratch_types=[
            pltpu.VMEM((2, block), jnp.int32),
            pltpu.VMEM((2, block, 16), F32),
            pltpu.VMEM_SHARED((n_nodes, 16), F32),
        ] + [pltpu.SemaphoreType.DMA] * 6,
    )
    def k(rows_hbm, idx_hbm, zeros_hbm, out_hbm, idx_v, rows_v, acc_s, *sems):
        si = sems[0:2]
        sr = sems[2:4]
        ss = sems[4:6]
        c = lax.axis_index("c")
        s = lax.axis_index("s")
        wid = s * _NC + c
        pltpu.sync_copy(zeros_hbm.at[s],
                        acc_s.at[pl.ds(s * rows_seg, rows_seg)])
        plsc.subcore_barrier()

        def base(i):
            return wid * (nb * block) + i * block

        loads = [None, None]
        scat = [None, None]
        loads[0] = (
            pltpu.async_copy(idx_hbm.at[pl.ds(base(0), block)],
                             idx_v.at[0], si[0]),
            pltpu.async_copy(rows_hbm.at[pl.ds(base(0), block)],
                             rows_v.at[0], sr[0]),
        )
        for i in range(nb):
            sl = i % 2
            nsl = (i + 1) % 2
            loads[sl][0].wait()
            loads[sl][1].wait()
            scat[sl] = pltpu.async_copy(rows_v.at[sl],
                                        acc_s.at[idx_v.at[sl]], ss[sl],
                                        add=True)
            if i + 1 < nb:
                if scat[nsl] is not None:
                    scat[nsl].wait()
                loads[nsl] = (
                    pltpu.async_copy(idx_hbm.at[pl.ds(base(i + 1), block)],
                                     idx_v.at[nsl], si[nsl]),
                    pltpu.async_copy(rows_hbm.at[pl.ds(base(i + 1), block)],
                                     rows_v.at[nsl], sr[nsl]),
                )
        scat[(nb - 1) % 2].wait()
        plsc.subcore_barrier()
        pltpu.sync_copy(acc_s.at[pl.ds(s * rows_seg, rows_seg)],
                        out_hbm.at[c, s])

    out = k(rows, idx, zeros.reshape(_NS, rows_seg, 16))
    return out.reshape(_NC, n_nodes, 16)


def _sc_edge_gather(tbl, nf16, src, dst, block):
    E = src.shape[0]
    NB = E // block
    nb_per_w = (NB + _NW - 1) // _NW

    @functools.partial(
        pl.kernel, mesh=_sc_mesh(), compiler_params=_SC_PARAMS,
        out_type=[
            jax.ShapeDtypeStruct((E, 32), F32),
            jax.ShapeDtypeStruct((E, 16), F32),
        ],
        scratch_types=[
            pltpu.VMEM((block,), jnp.int32),
            pltpu.VMEM((block, 32), F32),
            pltpu.VMEM((block, 16), F32),
        ],
    )
    def k(tbl_hbm, nf_hbm, src_hbm, dst_hbm, xs_hbm, xd_hbm, idx_v, ra_v, rb_v):
        c = lax.axis_index("c")
        s = lax.axis_index("s")
        wid = s * _NC + c

        def body(i, _):
            j = i * _NW + wid

            @pl.when(j < NB)
            def _():
                base = j * block
                pltpu.sync_copy(src_hbm.at[pl.ds(base, block)], idx_v)
                pltpu.sync_copy(tbl_hbm.at[idx_v], ra_v)
                pltpu.sync_copy(ra_v, xs_hbm.at[pl.ds(base, block)])
                pltpu.sync_copy(dst_hbm.at[pl.ds(base, block)], idx_v)
                pltpu.sync_copy(nf_hbm.at[idx_v], rb_v)
                pltpu.sync_copy(rb_v, xd_hbm.at[pl.ds(base, block)])

            return 0

        lax.fori_loop(0, nb_per_w, body, 0)

    return k(tbl, nf16, src, dst)


def _sc_edge_gather_pipe(tbl, nf16, src, dst, block):
    E = src.shape[0]
    nb = E // (_NW * block)

    @functools.partial(
        pl.kernel, mesh=_sc_mesh(), compiler_params=_SC_PARAMS,
        out_type=[
            jax.ShapeDtypeStruct((E, 32), F32),
            jax.ShapeDtypeStruct((E, 16), F32),
        ],
        scratch_types=[
            pltpu.VMEM((2, block), jnp.int32),
            pltpu.VMEM((2, block), jnp.int32),
            pltpu.VMEM((2, block, 32), F32),
            pltpu.VMEM((2, block, 16), F32),
        ] + [pltpu.SemaphoreType.DMA] * 12,
    )
    def k(tbl_hbm, nf_hbm, src_hbm, dst_hbm, xs_hbm, xd_hbm,
          isv, idv, ra_v, rb_v, *sems):
        ia = sems[0:2]
        ib = sems[2:4]
        ga = sems[4:6]
        gb = sems[6:8]
        wa = sems[8:10]
        wb = sems[10:12]
        c = lax.axis_index("c")
        s = lax.axis_index("s")
        wid = s * _NC + c

        def base(i):
            return wid * (nb * block) + i * block

        writes = [None, None]
        cp_ia = pltpu.async_copy(src_hbm.at[pl.ds(base(0), block)],
                                 isv.at[0], ia[0])
        cp_ib = pltpu.async_copy(dst_hbm.at[pl.ds(base(0), block)],
                                 idv.at[0], ib[0])
        idx_cp = [(cp_ia, cp_ib), None]
        for i in range(nb):
            sl = i % 2
            nsl = (i + 1) % 2
            idx_cp[sl][0].wait()
            idx_cp[sl][1].wait()
            if writes[sl] is not None:
                writes[sl][0].wait()
                writes[sl][1].wait()
            g1 = pltpu.async_copy(tbl_hbm.at[isv.at[sl]], ra_v.at[sl], ga[sl])
            g2 = pltpu.async_copy(nf_hbm.at[idv.at[sl]], rb_v.at[sl], gb[sl])
            if i + 1 < nb:
                idx_cp[nsl] = (
                    pltpu.async_copy(src_hbm.at[pl.ds(base(i + 1), block)],
                                     isv.at[nsl], ia[nsl]),
                    pltpu.async_copy(dst_hbm.at[pl.ds(base(i + 1), block)],
                                     idv.at[nsl], ib[nsl]),
                )
            g1.wait()
            g2.wait()
            writes[sl] = (
                pltpu.async_copy(ra_v.at[sl],
                                 xs_hbm.at[pl.ds(base(i), block)], wa[sl]),
                pltpu.async_copy(rb_v.at[sl],
                                 xd_hbm.at[pl.ds(base(i), block)], wb[sl]),
            )
        for w in writes:
            if w is not None:
                w[0].wait()
                w[1].wait()

    return k(tbl, nf16, src, dst)


def _sc_cell_reduce(rows, idx, zeros, negs, n_pad, block):
    E = rows.shape[0]
    NB = E // block
    seg = n_pad // _NW
    ngrp = block // 16

    @functools.partial(
        pl.kernel, mesh=_sc_mesh(), compiler_params=_SC_PARAMS,
        out_type=[
            jax.ShapeDtypeStruct((_NW, seg, 16), F32),
            jax.ShapeDtypeStruct((_NW, seg, 16), F32),
        ],
        scratch_types=[
            pltpu.VMEM((block,), jnp.int32),
            pltpu.VMEM((block, 16), F32),
            pltpu.VMEM((seg, 16), F32),
            pltpu.VMEM((seg, 16), F32),
        ],
    )
    def k(rows_hbm, idx_hbm, zeros_hbm, negs_hbm, sum_hbm, max_hbm,
          idx_v, rows_v, acc_sum, acc_max):
        c = lax.axis_index("c")
        s = lax.axis_index("s")
        wid = s * _NC + c
        lo = wid * seg
        hi = lo + seg
        pltpu.sync_copy(zeros_hbm, acc_sum)
        pltpu.sync_copy(negs_hbm, acc_max)

        def blk(j, _):
            base = j * block
            pltpu.sync_copy(idx_hbm.at[pl.ds(base, block)], idx_v)
            pltpu.sync_copy(rows_hbm.at[pl.ds(base, block)], rows_v)

            def grp(t, _):
                iv = idx_v[pl.ds(t * 16, 16)]
                hit = jnp.any((iv >= lo) & (iv < hi))

                @pl.when(hit)
                def _():
                    for e in range(16):
                        ii = iv[e]

                        @pl.when((ii >= lo) & (ii < hi))
                        def _():
                            r = ii - lo
                            row = rows_v[t * 16 + e]
                            acc_sum[r] = acc_sum[r] + row
                            acc_max[r] = jnp.maximum(acc_max[r], row)

                return 0

            lax.fori_loop(0, ngrp, grp, 0)
            return 0

        lax.fori_loop(0, NB, blk, 0)
        pltpu.sync_copy(acc_sum, sum_hbm.at[wid])
        pltpu.sync_copy(acc_max, max_hbm.at[wid])

    sm, mx = k(rows, idx, zeros, negs)
    return sm.reshape(n_pad, 16), mx.reshape(n_pad, 16)


def _net_mlp_body(xa_ref, xb_ref, wa_ref, wb_ref, b1_ref,
                  w2_ref, b2_ref, w3_ref, b3_ref, w4_ref, b4_ref,
                  w5_ref, b5_ref, out_ref):
    xa = xa_ref[...]
    xb = xb_ref[...]
    h = _leaky(jnp.dot(xa, wa_ref[...], preferred_element_type=F32)
               + jnp.dot(xb, wb_ref[...], preferred_element_type=F32)
               + b1_ref[...])
    h = _leaky(jnp.dot(h, w2_ref[...], preferred_element_type=F32) + b2_ref[...])
    h = _leaky(jnp.dot(h, w3_ref[...], preferred_element_type=F32) + b3_ref[...])
    h = _leaky(jnp.dot(h, w4_ref[...], preferred_element_type=F32) + b4_ref[...])
    out_ref[...] = jnp.dot(h, w5_ref[...], preferred_element_type=F32) + b5_ref[...]


def _diag4(w):
    return block_diag(w, w, w, w)


def _tile4(b):
    return jnp.tile(b, 4)[None]


def _run_net_mlp(xa, xb, ps):
    E = xa.shape[0]
    Ep = E // 4
    T = 1000 if Ep % 1000 == 0 else Ep
    (w1, b1), (w2, b2), (w3, b3), (w4, b4), (w5, b5) = ps
    wa = _diag4(_pad_rows(w1[0:28], 32))
    wb = _diag4(w1[28:44])
    w5p = _diag4(jnp.pad(w5, ((0, 0), (0, 4))))
    b5p = jnp.concatenate([b5, jnp.ones((1,), F32), jnp.zeros((3,), F32)])
    const = lambda shape: pl.BlockSpec(shape, lambda i: (0, 0))
    out = pl.pallas_call(
        _net_mlp_body,
        grid=(Ep // T,),
        in_specs=[
            pl.BlockSpec((T, 128), lambda i: (i, 0)),
            pl.BlockSpec((T, 64), lambda i: (i, 0)),
            const((128, 256)), const((64, 256)), const((1, 256)),
            const((256, 256)), const((1, 256)),
            const((256, 256)), const((1, 256)),
            const((256, 256)), const((1, 256)),
            const((256, 64)), const((1, 64)),
        ],
        out_specs=pl.BlockSpec((T, 64), lambda i: (i, 0)),
        out_shape=jax.ShapeDtypeStruct((Ep, 64), F32),
    )(xa.reshape(Ep, 128), xb.reshape(Ep, 64),
      wa, wb, _tile4(b1), _diag4(w2), _tile4(b2), _diag4(w3), _tile4(b3),
      _diag4(w4), _tile4(b4), w5p, _tile4(b5p))
    return out.reshape(E, 16)


def _cell1_body(xa_ref, xb_ref, ef_ref,
                wa_ref, wb_ref, b1_ref, w2_ref, b2_ref, w3_ref, b3_ref,
                w4_ref, b4_ref, w5_ref, b5_ref,
                wq_ref, bq_ref, wk_ref, bk_ref, wv_ref, bv_ref,
                qqv_ref, kv_ref, vec_ref):
    i = pl.program_id(0)
    xa = xa_ref[...]
    xb = xb_ref[...]
    h = _leaky(jnp.dot(xa, wa_ref[...], preferred_element_type=F32)
               + jnp.dot(xb, wb_ref[...], preferred_element_type=F32)
               + b1_ref[...])
    h = _leaky(jnp.dot(h, w2_ref[...], preferred_element_type=F32) + b2_ref[...])
    h = _leaky(jnp.dot(h, w3_ref[...], preferred_element_type=F32) + b3_ref[...])
    h = _leaky(jnp.dot(h, w4_ref[...], preferred_element_type=F32) + b4_ref[...])
    Q = (jnp.dot(h, w5_ref[...], preferred_element_type=F32) + b5_ref[...]
         + xa[:, 0:12])
    q = jnp.dot(Q, wq_ref[...], preferred_element_type=F32) + bq_ref[...]
    ef = ef_ref[...]
    k = jnp.dot(ef[:, 0:EF_KEY_], wk_ref[...], preferred_element_type=F32) + bk_ref[...]
    v = jnp.dot(ef[:, EF_KEY_:], wv_ref[...], preferred_element_type=F32) + bv_ref[...]
    T = q.shape[0]
    qqv_ref[...] = jnp.concatenate(
        [Q, q, v, jnp.zeros((T, 12), F32)], axis=1)
    kv = lax.dot_general(k, v, (((0,), (0,)), ((), ())),
                         preferred_element_type=F32)
    ksum = jnp.sum(k, axis=0)
    sq = jnp.sum(q * q)
    sk = jnp.sum(k * k)
    row = jnp.concatenate(
        [ksum, jnp.reshape(sq, (1,)), jnp.reshape(sk, (1,)),
         jnp.zeros((114,), F32)])[None]

    @pl.when(i == 0)
    def _():
        kv_ref[...] = jnp.zeros_like(kv_ref)
        vec_ref[...] = jnp.zeros_like(vec_ref)

    kv_ref[...] += kv
    vec_ref[...] += row


def _run_cell1(xa, xb, ef, ps_q, wq, bq, wk, bk, wv, bv):
    E = xa.shape[0]
    T = TE if E % TE == 0 else E
    (w1, b1), (w2, b2), (w3, b3), (w4, b4), (w5, b5) = ps_q
    wa = jnp.concatenate(
        [w1[32:44], w1[0:16], jnp.zeros((4, 64), F32)], axis=0)
    wb = w1[16:32]
    const = lambda shape: pl.BlockSpec(shape, lambda i: (0, 0))
    return pl.pallas_call(
        _cell1_body,
        grid=(E // T,),
        in_specs=[
            pl.BlockSpec((T, 32), lambda i: (i, 0)),
            pl.BlockSpec((T, 16), lambda i: (i, 0)),
            pl.BlockSpec((T, 512), lambda i: (i, 0)),
            const((32, 64)), const((16, 64)), const((1, 64)),
            const((64, 64)), const((1, 64)),
            const((64, 64)), const((1, 64)),
            const((64, 64)), const((1, 64)),
            const((64, 12)), const((1, 12)),
            const((12, 12)), const((1, 12)),
            const((120, 12)), const((1, 12)),
            const((392, 12)), const((1, 12)),
        ],
        out_specs=[
            pl.BlockSpec((T, 48), lambda i: (i, 0)),
            pl.BlockSpec((12, 12), lambda i: (0, 0)),
            pl.BlockSpec((1, 128), lambda i: (0, 0)),
        ],
        out_shape=[
            jax.ShapeDtypeStruct((E, 48), F32),
            jax.ShapeDtypeStruct((12, 12), F32),
            jax.ShapeDtypeStruct((1, 128), F32),
        ],
    )(xa, xb, ef, wa, wb, b1[None], w2, b2[None], w3, b3[None], w4, b4[None],
      w5, b5[None], wq, bq[None], wk, bk[None], wv, bv[None])


def _cell2_body(qqv_ref, xa_ref, xb_ref, kv_ref, vec_ref,
                wa_ref, wb_ref, wy_ref, b1_ref, w2_ref, b2_ref,
                w3_ref, b3_ref, w4_ref, b4_ref, w5_ref, b5_ref,
                g_ref, bb_ref, nedges_ref,
                mji_ref, efce_ref):
    qqv = qqv_ref[...]
    Q = qqv[:, 0:12]
    q = qqv[:, 12:24]
    v = qqv[:, 24:36]
    vec = vec_ref[...]
    ksum = vec[0, 0:12]
    sq = vec[0, 12]
    sk = vec[0, 13]
    NE = nedges_ref[0, 0]
    inv_nq = lax.rsqrt(sq)
    inv_nk = lax.rsqrt(sk)
    qn = q * inv_nq
    kv = kv_ref[...] * inv_nk
    ks = ksum * inv_nk
    att_num = jnp.dot(qn, kv, preferred_element_type=F32) + NE * v
    att_den = jnp.dot(qn, ks[:, None], preferred_element_type=F32) + NE
    O = att_num / att_den
    x = Q + O
    mu = jnp.mean(x, axis=1, keepdims=True)
    var = jnp.mean((x - mu) ** 2, axis=1, keepdims=True)
    y = (x - mu) * lax.rsqrt(var + 1e-5) * g_ref[...] + bb_ref[...]
    xa = xa_ref[...]
    xb = xb_ref[...]
    h = _leaky(jnp.dot(xa, wa_ref[...], preferred_element_type=F32)
               + jnp.dot(xb, wb_ref[...], preferred_element_type=F32)
               + jnp.dot(y, wy_ref[...], preferred_element_type=F32)
               + b1_ref[...])
    h = _leaky(jnp.dot(h, w2_ref[...], preferred_element_type=F32) + b2_ref[...])
    h = _leaky(jnp.dot(h, w3_ref[...], preferred_element_type=F32) + b3_ref[...])
    h = _leaky(jnp.dot(h, w4_ref[...], preferred_element_type=F32) + b4_ref[...])
    xm = jnp.dot(h, w5_ref[...], preferred_element_type=F32) + b5_ref[...]
    T = xm.shape[0]
    mji = xm[:, 0:12] + xa[:, 0:12]
    mji_ref[...] = jnp.concatenate(
        [mji, jnp.ones((T, 1), F32), jnp.zeros((T, 3), F32)], axis=1)
    efce_ref[...] = xm[:, 12:16]


def _run_cell2(qqv, xa, xb, kv, vec, ps_m, ln_g, ln_b):
    E = xa.shape[0]
    T = TE if E % TE == 0 else E
    (w1, b1), (w2, b2), (w3, b3), (w4, b4), (w5, b5) = ps_m
    wa = jnp.concatenate(
        [w1[32:44], w1[0:16], jnp.zeros((4, 64), F32)], axis=0)
    wb = w1[16:32]
    wy = w1[44:56]
    nedges = jnp.full((1, 1), float(E), F32)
    const = lambda shape: pl.BlockSpec(shape, lambda i: (0, 0))
    return pl.pallas_call(
        _cell2_body,
        grid=(E // T,),
        in_specs=[
            pl.BlockSpec((T, 48), lambda i: (i, 0)),
            pl.BlockSpec((T, 32), lambda i: (i, 0)),
            pl.BlockSpec((T, 16), lambda i: (i, 0)),
            const((12, 12)), const((1, 128)),
            const((32, 64)), const((16, 64)), const((12, 64)), const((1, 64)),
            const((64, 64)), const((1, 64)),
            const((64, 64)), const((1, 64)),
            const((64, 64)), const((1, 64)),
            const((64, 16)), const((1, 16)),
            const((1, 12)), const((1, 12)), const((1, 1)),
        ],
        out_specs=[
            pl.BlockSpec((T, 16), lambda i: (i, 0)),
            pl.BlockSpec((T, 4), lambda i: (i, 0)),
        ],
        out_shape=[
            jax.ShapeDtypeStruct((E, 16), F32),
            jax.ShapeDtypeStruct((E, 4), F32),
        ],
    )(qqv, xa, xb, kv, vec, wa, wb, wy, b1[None], w2, b2[None], w3, b3[None],
      w4, b4[None], w5, b5[None], ln_g[None], ln_b[None], nedges)


def _final_body(csum_ref, cmax_ref, nsum0_ref, nsum1_ref, pip0_ref, pip1_ref,
                lastnf_ref, nf_ref,
                wmean_ref, wmax_ref, wnf_ref, b1_ref, w2_ref, b2_ref,
                w3_ref, b3_ref, w4_ref, b4_ref, w5_ref, b5_ref,
                out_ref):
    csum = csum_ref[...]
    cnt = csum[:, 12:13]
    cmask = cnt > 0
    mean = csum[:, 0:12] / jnp.maximum(cnt, 1.0)
    mx = jnp.where(cmask, cmax_ref[:, 0:12], 0.0)
    nf = nf_ref[...]
    h = _leaky(jnp.dot(mean, wmean_ref[...], preferred_element_type=F32)
               + jnp.dot(mx, wmax_ref[...], preferred_element_type=F32)
               + jnp.dot(nf, wnf_ref[...], preferred_element_type=F32)
               + b1_ref[...])
    h = _leaky(jnp.dot(h, w2_ref[...], preferred_element_type=F32) + b2_ref[...])
    h = _leaky(jnp.dot(h, w3_ref[...], preferred_element_type=F32) + b3_ref[...])
    h = _leaky(jnp.dot(h, w4_ref[...], preferred_element_type=F32) + b4_ref[...])
    red = jnp.dot(h, w5_ref[...], preferred_element_type=F32) + b5_ref[...]
    nsum = nsum0_ref[0] + nsum1_ref[0]
    nmask = nsum[:, 12:13] > 0
    pmask = (pip0_ref[0, :, 12:13] + pip1_ref[0, :, 12:13]) > 0
    base = jnp.where(pmask, lastnf_ref[...], 0.0)
    out_ref[...] = jnp.where(cmask, red,
                             jnp.where(nmask, nsum[:, 0:12], base))


def _run_final(csum, cmax, nsum_parts, pi_parts, last_nf, nf, ps_a):
    N = nf.shape[0]
    T = TN if N % TN == 0 else N
    (w1, b1), (w2, b2), (w3, b3), (w4, b4), (w5, b5) = ps_a
    wmean = w1[0:12]
    wmax = w1[12:24]
    wnf = w1[24:40]
    const = lambda shape: pl.BlockSpec(shape, lambda i: (0, 0))
    return pl.pallas_call(
        _final_body,
        grid=(N // T,),
        in_specs=[
            pl.BlockSpec((T, 16), lambda i: (i, 0)),
            pl.BlockSpec((T, 16), lambda i: (i, 0)),
            pl.BlockSpec((1, T, 16), lambda i: (0, i, 0)),
            pl.BlockSpec((1, T, 16), lambda i: (1, i, 0)),
            pl.BlockSpec((1, T, 16), lambda i: (0, i, 0)),
            pl.BlockSpec((1, T, 16), lambda i: (1, i, 0)),
            pl.BlockSpec((T, 12), lambda i: (i, 0)),
            pl.BlockSpec((T, 16), lambda i: (i, 0)),
            const((12, 64)), const((12, 64)), const((16, 64)), const((1, 64)),
            const((64, 64)), const((1, 64)),
            const((64, 64)), const((1, 64)),
            const((64, 64)), const((1, 64)),
            const((64, 12)), const((1, 12)),
        ],
        out_specs=pl.BlockSpec((T, 12), lambda i: (i, 0)),
        out_shape=jax.ShapeDtypeStruct((N, 12), F32),
    )(csum, cmax, nsum_parts, nsum_parts, pi_parts, pi_parts, last_nf, nf,
      wmean, wmax, wnf, b1[None],
      w2, b2[None], w3, b3[None], w4, b4[None], w5, b5[None])


def kernel(nf, n_ats, n_slew_log, n_net_delays_log, ef, params,
           edge_index_net, edge_index_cell, pi_nodes):
    N = nf.shape[0]
    last_nf = jnp.concatenate([n_ats, n_slew_log, n_net_delays_log], axis=1)
    tbl = jnp.concatenate([last_nf, nf, jnp.zeros((N, 4), F32)], axis=1)
    zeros16 = jnp.zeros((N, 16), F32)

    src = edge_index_net[0].astype(jnp.int32)
    dst = edge_index_net[1].astype(jnp.int32)
    xa_n, xb_n = _sc_edge_gather(tbl, nf, src, dst, 1000)
    efn = _run_net_mlp(xa_n, xb_n, params['netprop'])
    net_parts = _sc_scatter_add_pipe(efn, dst, zeros16, N, 1000)

    pi32 = pi_nodes.astype(jnp.int32)
    pi_parts = _sc_scatter_add(
        jnp.ones((pi32.shape[0], 16), F32), pi32, zeros16, N, pi32.shape[0])

    s = edge_index_cell[0].astype(jnp.int32)
    d = edge_index_cell[1].astype(jnp.int32)
    xa_c, xb_c = _sc_edge_gather(tbl, nf, s, d, 1000)
    wq, bq = params['Wq']
    wk, bk = params['Wk']
    wv, bv = params['Wv']
    qqv, kv, vec = _run_cell1(xa_c, xb_c, ef, params['q'],
                              wq, bq, wk, bk, wv, bv)
    mji, efce = _run_cell2(qqv, xa_c, xb_c, kv, vec, params['m'],
                           params['ln_g'], params['ln_b'])
    NPAD = 50048
    seg = NPAD // 32
    csum_p, cmax_p = _sc_cell_reduce(
        mji, d, jnp.zeros((seg, 16), F32), jnp.full((seg, 16), -1e30, F32),
        NPAD, 800)
    cell_sum = csum_p[:N]
    cell_max = cmax_p[:N]

    new_nf = _run_final(cell_sum, cell_max, net_parts, pi_parts, last_nf, nf,
                        params['a'])
    return (new_nf, efce)
